# baseline (speedup 1.0000x reference)
import functools

import jax
import jax.numpy as jnp
from jax import lax
from jax.experimental import pallas as pl
from jax.experimental.pallas import tpu as pltpu
from jax.experimental.pallas import tpu_sc as plsc

T, N, D, H, G, OUT = 4, 10000, 128, 128, 16, 128
NC, NS = 2, 16
NW = NC * NS
CH = 128
N_PAD = 10240
TRASH = N
ZROWS = N_PAD // NS


def _front_body(x_ref, wnt_ref, bnt_ref, wat_ref, bat_ref, h_ref, aw_ref):
    x = x_ref[...]
    wnt = wnt_ref[...]
    wat = wat_ref[...]
    bnt = bnt_ref[...]
    bat = bat_ref[...]
    hs = []
    ats = []
    for t in range(T):
        ht = jnp.dot(x[t], wnt, preferred_element_type=jnp.float32) + bnt
        h_ref[t] = ht
        hs.append(ht)
        ats.append(jnp.dot(ht, wat, preferred_element_type=jnp.float32) + bat)
    for i in range(T - 1):
        sc = ats[i] * ats[i + 1]
        m = jnp.max(sc, axis=-1, keepdims=True)
        e = jnp.exp(sc - m)
        aw_ref[i] = e / jnp.sum(e, axis=-1, keepdims=True)


def _tc_front(x, w_nt, b_nt, w_attn, b_attn):
    nb = 400
    grid = (N // nb,)
    return pl.pallas_call(
        _front_body,
        grid=grid,
        in_specs=[
            pl.BlockSpec((T, nb, D), lambda n: (0, n, 0)),
            pl.BlockSpec((D, H), lambda n: (0, 0)),
            pl.BlockSpec((1, H), lambda n: (0, 0)),
            pl.BlockSpec((H, H), lambda n: (0, 0)),
            pl.BlockSpec((1, H), lambda n: (0, 0)),
        ],
        out_specs=[
            pl.BlockSpec((T, nb, H), lambda n: (0, n, 0)),
            pl.BlockSpec((T - 1, nb, H), lambda n: (0, n, 0)),
        ],
        out_shape=[
            jax.ShapeDtypeStruct((T, N, H), jnp.float32),
            jax.ShapeDtypeStruct((T - 1, N, H), jnp.float32),
        ],
    )(x, w_nt, b_nt.reshape(1, H), w_attn, b_attn.reshape(1, H))


CPW = 80
GC = 16
NG = CPW // GC


def _seg_body(aw_ref, dst_ref, src_ref, zeros_ref, parts_ref,
              idxd, idxs, rows, acc,
              id0, id1, is0, is1, gs0, gs1, ss0, ss1):
    c = lax.axis_index("c")
    s = lax.axis_index("s")
    wid = s * NC + c
    rows_per_snap = dst_ref.shape[0] // (T - 1)
    isem_d = [id0, id1]
    isem_s = [is0, is1]
    gsem = [gs0, gs1]
    ssem = [ss0, ss1]

    for i in range(T - 1):
        dbase = i * rows_per_snap + wid * CPW
        sbase = wid * CPW

        def load_idx(g, sync=False):
            p = g % 2
            if sync:
                pltpu.sync_copy(dst_ref.at[pl.ds(dbase + g * GC, GC)],
                                idxd.at[p])
                pltpu.sync_copy(src_ref.at[pl.ds(sbase + g * GC, GC)],
                                idxs.at[p])
                return None
            return (
                pltpu.async_copy(dst_ref.at[pl.ds(dbase + g * GC, GC)],
                                 idxd.at[p], isem_d[p]),
                pltpu.async_copy(src_ref.at[pl.ds(sbase + g * GC, GC)],
                                 idxs.at[p], isem_s[p]),
            )

        def gather(k):
            return pltpu.async_copy(
                aw_ref.at[idxd.at[(k // GC) % 2, k % GC]], rows.at[k % 2],
                gsem[k % 2])

        def scatter(k):
            return pltpu.async_copy(
                rows.at[k % 2], acc.at[idxs.at[(k // GC) % 2, k % GC]],
                ssem[k % 2], add=True)

        load_idx(0, sync=True)
        pltpu.sync_copy(zeros_ref, acc.at[pl.ds(s * ZROWS, ZROWS)])
        plsc.subcore_barrier()

        descs = {}
        descs[("g", 0)] = gather(0)
        for g in range(NG):
            for k in range(GC):
                ck = g * GC + k
                if ck > 0:
                    descs[("s", ck - 1)].wait()
                if k == 0 and g + 1 < NG:
                    descs[("i", g + 1)] = load_idx(g + 1)
                if ck + 1 < CPW:
                    if (ck + 1) % GC == 0:
                        for d in descs[("i", g + 1)]:
                            d.wait()
                    descs[("g", ck + 1)] = gather(ck + 1)
                descs[("g", ck)].wait()
                descs[("s", ck)] = scatter(ck)
        descs[("s", CPW - 1)].wait()

        plsc.subcore_barrier()
        pltpu.sync_copy(acc.at[pl.ds(s * ZROWS, ZROWS)],
                        parts_ref.at[c, i, pl.ds(s * ZROWS, ZROWS)])
        plsc.subcore_barrier()


def _sc_segsum(aw_flat, dst2d, src2d, zeros):
    mesh = plsc.VectorSubcoreMesh(
        core_axis_name="c", subcore_axis_name="s", num_cores=NC,
        num_subcores=NS)
    return pl.kernel(
        _seg_body,
        out_type=jax.ShapeDtypeStruct((NC, T - 1, N_PAD, H), jnp.float32),
        mesh=mesh,
        scratch_types=[
            pltpu.VMEM((2, GC, CH), jnp.int32),
            pltpu.VMEM((2, GC, CH), jnp.int32),
            pltpu.VMEM((2, CH, H), jnp.float32),
            pltpu.VMEM_SHARED((N_PAD, H), jnp.float32),
            pltpu.SemaphoreType.DMA,
            pltpu.SemaphoreType.DMA,
            pltpu.SemaphoreType.DMA,
            pltpu.SemaphoreType.DMA,
            pltpu.SemaphoreType.DMA,
            pltpu.SemaphoreType.DMA,
            pltpu.SemaphoreType.DMA,
            pltpu.SemaphoreType.DMA,
        ],
    )(aw_flat, dst2d, src2d, zeros)


def _back_body(h_ref, parts_ref, bidx_ref, wsda_ref, wsdb_ref, bsd_ref,
               wg1_ref, bg1_ref, wg2_ref, bg2_ref, wm1_ref, bm1_ref,
               wm2_ref, bm2_ref, out_ref, acc):
    n = pl.program_id(0)
    nsteps = pl.num_programs(0)

    @pl.when(n == 0)
    def _():
        acc[...] = jnp.zeros_like(acc)

    h = h_ref[...]
    segs = parts_ref[0] + parts_ref[1]
    bsd = bsd_ref[...]
    tot = None
    for i in range(T - 1):
        w = (h[i + 1] - h[i]) * segs[i]
        e = jax.nn.relu(
            jnp.dot(h[i], wsda_ref[...], preferred_element_type=jnp.float32)
            + jnp.dot(w, wsdb_ref[...], preferred_element_type=jnp.float32)
            + bsd)
        tot = e if tot is None else tot + e
    prop = tot * (1.0 / (T - 1))
    g1 = jax.nn.relu(
        jnp.dot(prop, wg1_ref[...], preferred_element_type=jnp.float32)
        + bg1_ref[...])
    gate = jax.nn.sigmoid(
        jnp.sum(g1 * wg2_ref[...], axis=-1, keepdims=True) + bg2_ref[0, 0])
    gp = gate * prop
    bidx = bidx_ref[0, 0]
    mask = (lax.broadcasted_iota(jnp.int32, (G, gp.shape[0]), 0)
            == bidx[None, :]).astype(jnp.float32)
    acc[...] += jnp.dot(mask, gp, preferred_element_type=jnp.float32)

    @pl.when(n == nsteps - 1)
    def _():
        ge = acc[...]
        o = jax.nn.relu(
            jnp.dot(ge, wm1_ref[...], preferred_element_type=jnp.float32)
            + bm1_ref[...])
        out_ref[...] = (
            jnp.dot(o, wm2_ref[...], preferred_element_type=jnp.float32)
            + bm2_ref[...])


def _tc_back(h, parts, bidx3, w_sd, b_sd, w_g1, b_g1, w_g2, b_g2,
             w_m1, b_m1, w_m2, b_m2):
    nb = 400
    nblk = N // nb
    return pl.pallas_call(
        _back_body,
        grid=(nblk,),
        in_specs=[
            pl.BlockSpec((T, nb, H), lambda n: (0, n, 0)),
            pl.BlockSpec((NC, T - 1, nb, H), lambda n: (0, 0, n, 0)),
            pl.BlockSpec((1, 1, nb), lambda n: (n, 0, 0)),
            pl.BlockSpec((H, H), lambda n: (0, 0)),
            pl.BlockSpec((H, H), lambda n: (0, 0)),
            pl.BlockSpec((1, H), lambda n: (0, 0)),
            pl.BlockSpec((H, H), lambda n: (0, 0)),
            pl.BlockSpec((1, H), lambda n: (0, 0)),
            pl.BlockSpec((1, H), lambda n: (0, 0)),
            pl.BlockSpec((1, 1), lambda n: (0, 0)),
            pl.BlockSpec((H, H), lambda n: (0, 0)),
            pl.BlockSpec((1, H), lambda n: (0, 0)),
            pl.BlockSpec((H, OUT), lambda n: (0, 0)),
            pl.BlockSpec((1, OUT), lambda n: (0, 0)),
        ],
        out_specs=pl.BlockSpec((G, OUT), lambda n: (0, 0)),
        out_shape=jax.ShapeDtypeStruct((G, OUT), jnp.float32),
        scratch_shapes=[pltpu.VMEM((G, H), jnp.float32)],
    )(h, parts, bidx3, w_sd[:H], w_sd[H:], b_sd.reshape(1, H),
      w_g1, b_g1.reshape(1, H), w_g2.reshape(1, H), b_g2.reshape(1, 1),
      w_m1, b_m1.reshape(1, H), w_m2, b_m2.reshape(1, OUT))


def kernel(x, edge_index, batch_idx, W_nt, b_nt, W_attn, b_attn, W_sd, b_sd,
           W_g1, b_g1, W_g2, b_g2, W_m1, b_m1, W_m2, b_m2):
    src = edge_index[0].astype(jnp.int32)
    dst = edge_index[1].astype(jnp.int32)
    e = src.shape[0]
    grp = NW * CPW * CH
    e_pad = ((e + grp - 1) // grp) * grp
    src_pad = jnp.concatenate(
        [src, jnp.full((e_pad - e,), TRASH, jnp.int32)])
    dst_pad = jnp.concatenate([dst, jnp.zeros((e_pad - e,), jnp.int32)])
    dst_all = (dst_pad[None, :]
               + (jnp.arange(T - 1, dtype=jnp.int32) * N)[:, None])
    dst2d = dst_all.reshape((T - 1) * e_pad // CH, CH)
    src2d = src_pad.reshape(e_pad // CH, CH)
    zeros = jnp.zeros((ZROWS, H), jnp.float32)

    h, aw = _tc_front(x, W_nt, b_nt, W_attn, b_attn)
    parts = _sc_segsum(aw.reshape((T - 1) * N, H), dst2d, src2d, zeros)
    return _tc_back(h, parts, batch_idx.astype(jnp.int32).reshape(N // 400, 1, 400),
                    W_sd, b_sd, W_g1, b_g1, W_g2, b_g2, W_m1, b_m1, W_m2, b_m2)


# speedup vs baseline: 1.0017x; 1.0017x over previous
import functools

import jax
import jax.numpy as jnp
from jax import lax
from jax.experimental import pallas as pl
from jax.experimental.pallas import tpu as pltpu
from jax.experimental.pallas import tpu_sc as plsc

T, N, D, H, G, OUT = 4, 10000, 128, 128, 16, 128
NC, NS = 2, 16
NW = NC * NS
CH = 128
N_PAD = 10240
TRASH = N
ZROWS = N_PAD // NS


def _front_body(x_ref, wnt_ref, bnt_ref, wat_ref, bat_ref, h_ref, aw_ref):
    x = x_ref[...]
    wnt = wnt_ref[...]
    wat = wat_ref[...]
    bnt = bnt_ref[...]
    bat = bat_ref[...]
    hs = []
    ats = []
    for t in range(T):
        ht = jnp.dot(x[t], wnt, preferred_element_type=jnp.float32) + bnt
        h_ref[t] = ht
        hs.append(ht)
        ats.append(jnp.dot(ht, wat, preferred_element_type=jnp.float32) + bat)
    for i in range(T - 1):
        sc = ats[i] * ats[i + 1]
        m = jnp.max(sc, axis=-1, keepdims=True)
        e = jnp.exp(sc - m)
        aw_ref[i] = e / jnp.sum(e, axis=-1, keepdims=True)


def _tc_front(x, w_nt, b_nt, w_attn, b_attn):
    nb = 400
    grid = (N // nb,)
    return pl.pallas_call(
        _front_body,
        grid=grid,
        in_specs=[
            pl.BlockSpec((T, nb, D), lambda n: (0, n, 0)),
            pl.BlockSpec((D, H), lambda n: (0, 0)),
            pl.BlockSpec((1, H), lambda n: (0, 0)),
            pl.BlockSpec((H, H), lambda n: (0, 0)),
            pl.BlockSpec((1, H), lambda n: (0, 0)),
        ],
        out_specs=[
            pl.BlockSpec((T, nb, H), lambda n: (0, n, 0)),
            pl.BlockSpec((T - 1, nb, H), lambda n: (0, n, 0)),
        ],
        out_shape=[
            jax.ShapeDtypeStruct((T, N, H), jnp.float32),
            jax.ShapeDtypeStruct((T - 1, N, H), jnp.float32),
        ],
    )(x, w_nt, b_nt.reshape(1, H), w_attn, b_attn.reshape(1, H))


CPW = 80
GC = 8
ITC = 16


def _seg_body(aw_ref, dst_ref, src_ref, zeros_ref, parts_ref,
              idxd, idxs, rows, acc,
              id0, id1, is0, is1, gs0, gs1, ss0, ss1):
    c = lax.axis_index("c")
    s = lax.axis_index("s")
    wid = s * NC + c
    rows_per_snap = dst_ref.shape[0] // (T - 1)
    isem_d = [id0, id1]
    isem_s = [is0, is1]
    gsem = [gs0, gs1]
    ssem = [ss0, ss1]

    def w_g(p):
        pltpu.make_async_copy(aw_ref.at[pl.ds(0, CH)], rows.at[p],
                              gsem[p]).wait()

    def w_s(p):
        pltpu.make_async_copy(aw_ref.at[pl.ds(0, CH)], rows.at[p],
                              ssem[p]).wait()

    def w_i(p):
        pltpu.make_async_copy(dst_ref.at[pl.ds(0, GC)], idxd.at[p],
                              isem_d[p]).wait()
        pltpu.make_async_copy(src_ref.at[pl.ds(0, GC)], idxs.at[p],
                              isem_s[p]).wait()

    def snap(i, carry):
        dbase = i * rows_per_snap + wid * CPW
        sbase = wid * CPW

        def pf(g, p):
            pltpu.async_copy(dst_ref.at[pl.ds(dbase + g * GC, GC)],
                             idxd.at[p], isem_d[p])
            pltpu.async_copy(src_ref.at[pl.ds(sbase + g * GC, GC)],
                             idxs.at[p], isem_s[p])

        def gather(pos, gp, bp):
            pltpu.async_copy(aw_ref.at[idxd.at[gp, pos]], rows.at[bp],
                             gsem[bp])

        def scatter(pos, gp, bp):
            pltpu.async_copy(rows.at[bp], acc.at[idxs.at[gp, pos]],
                             ssem[bp], add=True)

        pltpu.sync_copy(dst_ref.at[pl.ds(dbase, GC)], idxd.at[0])
        pltpu.sync_copy(src_ref.at[pl.ds(sbase, GC)], idxs.at[0])
        pltpu.sync_copy(zeros_ref, acc.at[pl.ds(s * ZROWS, ZROWS)])
        plsc.subcore_barrier()
        gather(0, 0, 0)

        def chunk_pos(k, last_gather, pf_after=None):
            if k > 0:
                w_s((k + 1) % 2)
            if pf_after is not None:
                pf_after()
            if k == GC - 1:
                w_i(1)
            if k == ITC - 1 and not last_gather:
                w_i(0)
            if not (k == ITC - 1 and last_gather):
                np1 = k + 1
                gather(np1 % GC, (np1 // GC) % 2, np1 % 2)
            w_g(k % 2)
            scatter(k % GC, (k // GC) % 2, k % 2)

        def body(t, carry2):
            @pl.when(t > 0)
            def _():
                w_s(1)
            pf(2 * t + 1, 1)
            for k in range(ITC):
                chunk_pos(k, last_gather=False,
                          pf_after=(lambda: pf(2 * t + 2, 0)) if k == GC
                          else None)
            return carry2

        lax.fori_loop(0, CPW // ITC - 1, body, 0)

        w_s(1)
        pf(CPW // GC - 1, 1)
        for k in range(ITC):
            chunk_pos(k, last_gather=True)
        w_s(1)

        plsc.subcore_barrier()
        pltpu.sync_copy(acc.at[pl.ds(s * ZROWS, ZROWS)],
                        parts_ref.at[c, i, pl.ds(s * ZROWS, ZROWS)])
        plsc.subcore_barrier()
        return carry

    lax.fori_loop(0, T - 1, snap, 0)


def _sc_segsum(aw_flat, dst2d, src2d, zeros):
    mesh = plsc.VectorSubcoreMesh(
        core_axis_name="c", subcore_axis_name="s", num_cores=NC,
        num_subcores=NS)
    return pl.kernel(
        _seg_body,
        out_type=jax.ShapeDtypeStruct((NC, T - 1, N_PAD, H), jnp.float32),
        mesh=mesh,
        scratch_types=[
            pltpu.VMEM((2, GC, CH), jnp.int32),
            pltpu.VMEM((2, GC, CH), jnp.int32),
            pltpu.VMEM((2, CH, H), jnp.float32),
            pltpu.VMEM_SHARED((N_PAD, H), jnp.float32),
            pltpu.SemaphoreType.DMA,
            pltpu.SemaphoreType.DMA,
            pltpu.SemaphoreType.DMA,
            pltpu.SemaphoreType.DMA,
            pltpu.SemaphoreType.DMA,
            pltpu.SemaphoreType.DMA,
            pltpu.SemaphoreType.DMA,
            pltpu.SemaphoreType.DMA,
        ],
    )(aw_flat, dst2d, src2d, zeros)


def _back_body(h_ref, parts_ref, bidx_ref, wsda_ref, wsdb_ref, bsd_ref,
               wg1_ref, bg1_ref, wg2_ref, bg2_ref, wm1_ref, bm1_ref,
               wm2_ref, bm2_ref, out_ref, acc):
    n = pl.program_id(0)
    nsteps = pl.num_programs(0)

    @pl.when(n == 0)
    def _():
        acc[...] = jnp.zeros_like(acc)

    h = h_ref[...]
    segs = parts_ref[0] + parts_ref[1]
    bsd = bsd_ref[...]
    tot = None
    for i in range(T - 1):
        w = (h[i + 1] - h[i]) * segs[i]
        e = jax.nn.relu(
            jnp.dot(h[i], wsda_ref[...], preferred_element_type=jnp.float32)
            + jnp.dot(w, wsdb_ref[...], preferred_element_type=jnp.float32)
            + bsd)
        tot = e if tot is None else tot + e
    prop = tot * (1.0 / (T - 1))
    g1 = jax.nn.relu(
        jnp.dot(prop, wg1_ref[...], preferred_element_type=jnp.float32)
        + bg1_ref[...])
    gate = jax.nn.sigmoid(
        jnp.sum(g1 * wg2_ref[...], axis=-1, keepdims=True) + bg2_ref[0, 0])
    gp = gate * prop
    bidx = bidx_ref[0, 0]
    mask = (lax.broadcasted_iota(jnp.int32, (G, gp.shape[0]), 0)
            == bidx[None, :]).astype(jnp.float32)
    acc[...] += jnp.dot(mask, gp, preferred_element_type=jnp.float32)

    @pl.when(n == nsteps - 1)
    def _():
        ge = acc[...]
        o = jax.nn.relu(
            jnp.dot(ge, wm1_ref[...], preferred_element_type=jnp.float32)
            + bm1_ref[...])
        out_ref[...] = (
            jnp.dot(o, wm2_ref[...], preferred_element_type=jnp.float32)
            + bm2_ref[...])


def _tc_back(h, parts, bidx3, w_sd, b_sd, w_g1, b_g1, w_g2, b_g2,
             w_m1, b_m1, w_m2, b_m2):
    nb = 400
    nblk = N // nb
    return pl.pallas_call(
        _back_body,
        grid=(nblk,),
        in_specs=[
            pl.BlockSpec((T, nb, H), lambda n: (0, n, 0)),
            pl.BlockSpec((NC, T - 1, nb, H), lambda n: (0, 0, n, 0)),
            pl.BlockSpec((1, 1, nb), lambda n: (n, 0, 0)),
            pl.BlockSpec((H, H), lambda n: (0, 0)),
            pl.BlockSpec((H, H), lambda n: (0, 0)),
            pl.BlockSpec((1, H), lambda n: (0, 0)),
            pl.BlockSpec((H, H), lambda n: (0, 0)),
            pl.BlockSpec((1, H), lambda n: (0, 0)),
            pl.BlockSpec((1, H), lambda n: (0, 0)),
            pl.BlockSpec((1, 1), lambda n: (0, 0)),
            pl.BlockSpec((H, H), lambda n: (0, 0)),
            pl.BlockSpec((1, H), lambda n: (0, 0)),
            pl.BlockSpec((H, OUT), lambda n: (0, 0)),
            pl.BlockSpec((1, OUT), lambda n: (0, 0)),
        ],
        out_specs=pl.BlockSpec((G, OUT), lambda n: (0, 0)),
        out_shape=jax.ShapeDtypeStruct((G, OUT), jnp.float32),
        scratch_shapes=[pltpu.VMEM((G, H), jnp.float32)],
    )(h, parts, bidx3, w_sd[:H], w_sd[H:], b_sd.reshape(1, H),
      w_g1, b_g1.reshape(1, H), w_g2.reshape(1, H), b_g2.reshape(1, 1),
      w_m1, b_m1.reshape(1, H), w_m2, b_m2.reshape(1, OUT))


def kernel(x, edge_index, batch_idx, W_nt, b_nt, W_attn, b_attn, W_sd, b_sd,
           W_g1, b_g1, W_g2, b_g2, W_m1, b_m1, W_m2, b_m2):
    src = edge_index[0].astype(jnp.int32)
    dst = edge_index[1].astype(jnp.int32)
    e = src.shape[0]
    grp = NW * CPW * CH
    e_pad = ((e + grp - 1) // grp) * grp
    src_pad = jnp.concatenate(
        [src, jnp.full((e_pad - e,), TRASH, jnp.int32)])
    dst_pad = jnp.concatenate([dst, jnp.zeros((e_pad - e,), jnp.int32)])
    dst_all = (dst_pad[None, :]
               + (jnp.arange(T - 1, dtype=jnp.int32) * N)[:, None])
    dst2d = dst_all.reshape((T - 1) * e_pad // CH, CH)
    src2d = src_pad.reshape(e_pad // CH, CH)
    zeros = jnp.zeros((ZROWS, H), jnp.float32)

    h, aw = _tc_front(x, W_nt, b_nt, W_attn, b_attn)
    parts = _sc_segsum(aw.reshape((T - 1) * N, H), dst2d, src2d, zeros)
    return _tc_back(h, parts, batch_idx.astype(jnp.int32).reshape(N // 400, 1, 400),
                    W_sd, b_sd, W_g1, b_g1, W_g2, b_g2, W_m1, b_m1, W_m2, b_m2)
